# Initial kernel scaffold; baseline (speedup 1.0000x reference)
import jax
import jax.numpy as jnp
from jax.experimental import pallas as pl


def kernel(x, table, gamma, beta):
    raise NotImplementedError("write your pallas kernel here")


# speedup vs baseline: 5.1444x; 5.1444x over previous
import functools
import math

import jax
import jax.numpy as jnp
from jax import lax
from jax.experimental import pallas as pl
from jax.experimental.pallas import tpu as pltpu
from jax.experimental.pallas import tpu_sc as plsc

VOCAB = 100000
HID = 64
PAD = -100


_LN_ROWS = 2000


def _ln_body(t_ref, g_ref, b_ref, o_ref):
    h = t_ref[:] * math.sqrt(float(HID))
    m = jnp.mean(h, axis=1, keepdims=True)
    d = h - m
    v = jnp.mean(d * d, axis=1, keepdims=True)
    o_ref[:] = d * lax.rsqrt(v + 1e-5) * g_ref[:] + b_ref[:]


def _normalize_table(table, gamma, beta):
    g2 = gamma.reshape(1, HID)
    b2 = beta.reshape(1, HID)
    return pl.pallas_call(
        _ln_body,
        grid=(VOCAB // _LN_ROWS,),
        in_specs=[
            pl.BlockSpec((_LN_ROWS, HID), lambda i: (i, 0)),
            pl.BlockSpec((1, HID), lambda i: (0, 0)),
            pl.BlockSpec((1, HID), lambda i: (0, 0)),
        ],
        out_specs=pl.BlockSpec((_LN_ROWS, HID), lambda i: (i, 0)),
        out_shape=jax.ShapeDtypeStruct((VOCAB, HID), jnp.float32),
    )(table, g2, b2)


_NC = 2
_NS = 16
_NW = _NC * _NS
_B = 16384 * 50
_PER_W = _B // _NW
_CH = 128
_NCH = _PER_W // _CH


@functools.partial(
    pl.kernel,
    mesh=plsc.VectorSubcoreMesh(core_axis_name="c", subcore_axis_name="s"),
    out_type=jax.ShapeDtypeStruct((_B, HID), jnp.float32),
    scratch_types=[
        pltpu.VMEM((_NCH, _CH), jnp.int32),
        pltpu.VMEM((_CH, HID), jnp.float32),
        pltpu.SemaphoreType.DMA,
    ],
    compiler_params=pltpu.CompilerParams(use_tc_tiling_on_sc=False),
)
def _gather_k(idx_hbm, tab_hbm, out_hbm, idx_v, rows_v, sem):
    wid = lax.axis_index("s") * _NC + lax.axis_index("c")
    pltpu.sync_copy(idx_hbm.at[wid], idx_v)

    def body(j, carry):
        pltpu.async_copy(tab_hbm.at[idx_v.at[j]], rows_v, sem).wait()
        pltpu.sync_copy(rows_v, out_hbm.at[pl.ds(wid * _PER_W + j * _CH, _CH)])
        return carry

    lax.fori_loop(0, _NCH, body, 0)


def kernel(x, table, gamma, beta):
    table_n = _normalize_table(table, gamma, beta)
    x_mapped = jnp.where(x == PAD, VOCAB - 100, x)
    x_mapped = jnp.clip(x_mapped, 0, VOCAB - 1)
    idx3 = x_mapped.reshape(_NW, _NCH, _CH)
    out = _gather_k(idx3, table_n)
    return out.reshape(16384, 50, HID)


# speedup vs baseline: 6.0062x; 1.1675x over previous
import functools
import math

import jax
import jax.numpy as jnp
from jax import lax
from jax.experimental import pallas as pl
from jax.experimental.pallas import tpu as pltpu
from jax.experimental.pallas import tpu_sc as plsc

VOCAB = 100000
HID = 64
PAD = -100


_LN_ROWS = 2000


def _ln_body(t_ref, g_ref, b_ref, o_ref):
    h = t_ref[:] * math.sqrt(float(HID))
    m = jnp.mean(h, axis=1, keepdims=True)
    d = h - m
    v = jnp.mean(d * d, axis=1, keepdims=True)
    o_ref[:] = d * lax.rsqrt(v + 1e-5) * g_ref[:] + b_ref[:]


def _normalize_table(table, gamma, beta):
    g2 = gamma.reshape(1, HID)
    b2 = beta.reshape(1, HID)
    return pl.pallas_call(
        _ln_body,
        grid=(VOCAB // _LN_ROWS,),
        in_specs=[
            pl.BlockSpec((_LN_ROWS, HID), lambda i: (i, 0)),
            pl.BlockSpec((1, HID), lambda i: (0, 0)),
            pl.BlockSpec((1, HID), lambda i: (0, 0)),
        ],
        out_specs=pl.BlockSpec((_LN_ROWS, HID), lambda i: (i, 0)),
        out_shape=jax.ShapeDtypeStruct((VOCAB, HID), jnp.float32),
    )(table, g2, b2)


_NC = 2
_NS = 16
_NW = _NC * _NS
_B = 16384 * 50
_PER_W = _B // _NW
_CH = 128
_NCH = _PER_W // _CH
_SUB = 2
_ROWS2 = _SUB * _CH
_NBUF = 4
_NCH2 = _PER_W // _ROWS2
_NROUND = _NCH2 // _NBUF - 1


@functools.partial(
    pl.kernel,
    mesh=plsc.VectorSubcoreMesh(core_axis_name="c", subcore_axis_name="s"),
    out_type=jax.ShapeDtypeStruct((_B, HID), jnp.float32),
    scratch_types=[
        pltpu.VMEM((_NCH, _CH), jnp.int32),
        [pltpu.VMEM((_ROWS2, HID), jnp.float32) for _ in range(_NBUF)],
        [pltpu.SemaphoreType.DMA for _ in range(_NBUF)],
        [pltpu.SemaphoreType.DMA for _ in range(_NBUF)],
    ],
    compiler_params=pltpu.CompilerParams(use_tc_tiling_on_sc=False),
)
def _gather_k(idx_hbm, tab_hbm, out_hbm, idx_v, bufs, gsems, wsems):
    wid = lax.axis_index("s") * _NC + lax.axis_index("c")
    base = wid * _PER_W
    pltpu.sync_copy(idx_hbm.at[wid], idx_v)

    def fire(g, b):
        for s in range(_SUB):
            pltpu.async_copy(
                tab_hbm.at[idx_v.at[g * _SUB + s]],
                bufs[b].at[pl.ds(s * _CH, _CH)],
                gsems[b],
            )

    def drain_gather(b):
        for s in range(_SUB):
            pltpu.make_async_copy(
                tab_hbm.at[idx_v.at[0]], bufs[b].at[pl.ds(s * _CH, _CH)], gsems[b]
            ).wait()

    def write(g, b):
        pltpu.async_copy(bufs[b], out_hbm.at[pl.ds(base + g * _ROWS2, _ROWS2)], wsems[b])

    def drain_write(b):
        pltpu.make_async_copy(
            bufs[b], out_hbm.at[pl.ds(base, _ROWS2)], wsems[b]
        ).wait()

    for b in range(_NBUF):
        fire(b, b)

    def round_body(p, carry):
        for b in range(_NBUF):
            drain_gather(b)
            write(p * _NBUF + b, b)
        for b in range(_NBUF):
            drain_write(b)
            fire((p + 1) * _NBUF + b, b)
        return carry

    lax.fori_loop(0, _NROUND, round_body, 0)

    for b in range(_NBUF):
        drain_gather(b)
        write(_NROUND * _NBUF + b, b)
    for b in range(_NBUF):
        drain_write(b)


def kernel(x, table, gamma, beta):
    table_n = _normalize_table(table, gamma, beta)
    x_mapped = jnp.where(x == PAD, VOCAB - 100, x)
    x_mapped = jnp.clip(x_mapped, 0, VOCAB - 1)
    idx3 = x_mapped.reshape(_NW, _NCH, _CH)
    out = _gather_k(idx3, table_n)
    return out.reshape(16384, 50, HID)
